# baseline (speedup 1.0000x reference)
import functools

import jax
import jax.numpy as jnp
from jax import lax
from jax.experimental import pallas as pl
from jax.experimental.pallas import tpu as pltpu
from jax.experimental.pallas import tpu_sc as plsc

_VOCAB = 1000000
_HIDDEN = 32
_OUT = 16
_PACK = 8
_NC, _NS = 2, 16


def _project_body(emb_ref, wwide_ref, bbig_ref, mask_ref, out_ref):
    h = emb_ref[...].astype(jnp.bfloat16)
    ww = wwide_ref[...]
    res_all = jnp.dot(h, ww, preferred_element_type=jnp.float32)
    block = res_all.shape[0]
    r3 = res_all.reshape(block // _PACK, _PACK, _PACK * _OUT)
    picked = r3 * mask_ref[...][None, :, :]
    out_ref[...] = jnp.sum(picked, axis=1) + bbig_ref[...]


def _project(emb, w, b_big, block_rows):
    n_rows = emb.shape[0]
    w_wide = jnp.tile(w, (1, _PACK)).astype(jnp.bfloat16)
    lane = jax.lax.broadcasted_iota(jnp.int32, (_PACK, _PACK * _OUT), 1)
    sub = jax.lax.broadcasted_iota(jnp.int32, (_PACK, _PACK * _OUT), 0)
    mask = (lane // _OUT == sub).astype(jnp.float32)
    grid = (n_rows // block_rows,)
    return pl.pallas_call(
        _project_body,
        grid=grid,
        in_specs=[
            pl.BlockSpec((block_rows, _HIDDEN), lambda i: (i, 0)),
            pl.BlockSpec((_HIDDEN, _PACK * _OUT), lambda i: (0, 0)),
            pl.BlockSpec((1, _PACK * _OUT), lambda i: (0, 0)),
            pl.BlockSpec((_PACK, _PACK * _OUT), lambda i: (0, 0)),
        ],
        out_specs=pl.BlockSpec((block_rows // _PACK, _PACK * _OUT), lambda i: (i, 0)),
        out_shape=jax.ShapeDtypeStruct((n_rows // _PACK, _PACK * _OUT), jnp.float32),
    )(emb, w_wide, b_big, mask)


def _sc_gather(table, idx):
    num_idx = idx.shape[0]
    nw = _NC * _NS
    b_per_w = num_idx // nw
    chunk = 3328

    mesh = plsc.VectorSubcoreMesh(core_axis_name="c", subcore_axis_name="s")

    @functools.partial(
        pl.kernel,
        mesh=mesh,
        out_type=jax.ShapeDtypeStruct((num_idx, _OUT), jnp.float32),
        compiler_params=pltpu.CompilerParams(use_tc_tiling_on_sc=False),
        scratch_types=[
            pltpu.VMEM((b_per_w,), jnp.int32),
            pltpu.VMEM((chunk, _OUT), jnp.float32),
            pltpu.SemaphoreType.DMA,
        ],
    )
    def gather_kernel(p_hbm, idx_hbm, out_hbm, idx_v, rows_v, sem):
        wid = lax.axis_index("s") * _NC + lax.axis_index("c")
        base = wid * b_per_w
        pltpu.sync_copy(idx_hbm.at[pl.ds(base, b_per_w)], idx_v)

        @pl.loop(0, b_per_w, step=chunk)
        def _(i):
            pltpu.async_copy(
                p_hbm.at[idx_v.at[pl.ds(i, chunk)]], rows_v, sem
            ).wait()
            pltpu.sync_copy(rows_v, out_hbm.at[pl.ds(base + i, chunk)])

    return gather_kernel(table, idx)


def kernel(x, emb, W, b):
    batch, fields = x.shape
    idx = x.reshape(-1).astype(jnp.int32)
    b_big = jnp.tile(b, _PACK).reshape(1, _PACK * _OUT)
    p_packed = _project(emb, W, b_big, block_rows=40000)
    p = p_packed.reshape(_VOCAB, _OUT)
    out = _sc_gather(p, idx)
    return out.reshape(batch, fields, _OUT)


# speedup vs baseline: 1.1375x; 1.1375x over previous
import functools

import jax
import jax.numpy as jnp
from jax import lax
from jax.experimental import pallas as pl
from jax.experimental.pallas import tpu as pltpu
from jax.experimental.pallas import tpu_sc as plsc

_HIDDEN = 32
_OUT = 16
_G4 = 4
_NC, _NS = 2, 16


def _sc_gather(table, idx):
    num_idx = idx.shape[0]
    d = table.shape[1]
    nw = _NC * _NS
    b_per_w = num_idx // nw
    chunk = 3328

    mesh = plsc.VectorSubcoreMesh(core_axis_name="c", subcore_axis_name="s")

    @functools.partial(
        pl.kernel,
        mesh=mesh,
        out_type=jax.ShapeDtypeStruct((num_idx, d), jnp.float32),
        compiler_params=pltpu.CompilerParams(use_tc_tiling_on_sc=False),
        scratch_types=[
            pltpu.VMEM((b_per_w,), jnp.int32),
            pltpu.VMEM((chunk, d), jnp.float32),
            pltpu.SemaphoreType.DMA,
        ],
    )
    def gather_kernel(table_hbm, idx_hbm, out_hbm, idx_v, rows_v, sem):
        wid = lax.axis_index("s") * _NC + lax.axis_index("c")
        base = wid * b_per_w
        pltpu.sync_copy(idx_hbm.at[pl.ds(base, b_per_w)], idx_v)

        @pl.loop(0, b_per_w, step=chunk)
        def _(i):
            pltpu.async_copy(
                table_hbm.at[idx_v.at[pl.ds(i, chunk)]], rows_v, sem
            ).wait()
            pltpu.sync_copy(rows_v, out_hbm.at[pl.ds(base + i, chunk)])

    return gather_kernel(table, idx)


def _project_body(g_ref, w2_ref, bbig_ref, mask_ref, out_ref):
    res = jnp.dot(g_ref[...], w2_ref[...], preferred_element_type=jnp.float32)
    half = res.shape[0] // 2
    r3 = res.reshape(half, 2, 128)
    out_ref[...] = jnp.sum(r3 * mask_ref[...][None, :, :], axis=1) + bbig_ref[...]


def _project(g4, w, b, block_rows):
    n_rows = g4.shape[0]
    w4 = jnp.kron(jnp.eye(_G4, dtype=w.dtype), w)
    w2 = jnp.concatenate([w4, w4], axis=1)
    b_big = jnp.tile(b, 8).reshape(1, 128)
    lane = lax.broadcasted_iota(jnp.int32, (2, 128), 1)
    sub = lax.broadcasted_iota(jnp.int32, (2, 128), 0)
    mask = (lane // 64 == sub).astype(jnp.float32)
    grid = (n_rows // block_rows,)
    return pl.pallas_call(
        _project_body,
        grid=grid,
        in_specs=[
            pl.BlockSpec((block_rows, 128), lambda i: (i, 0)),
            pl.BlockSpec((128, 128), lambda i: (0, 0)),
            pl.BlockSpec((1, 128), lambda i: (0, 0)),
            pl.BlockSpec((2, 128), lambda i: (0, 0)),
        ],
        out_specs=pl.BlockSpec((block_rows // 2, 128), lambda i: (i, 0)),
        out_shape=jax.ShapeDtypeStruct((n_rows // 2, 128), jnp.float32),
    )(g4, w2, b_big, mask)


def kernel(x, emb, W, b):
    batch, fields = x.shape
    idx = x.reshape(-1).astype(jnp.int32)
    g = _sc_gather(emb, idx)
    g4 = g.reshape(batch * fields // _G4, _G4 * _HIDDEN)
    out_packed = _project(g4, W, b, block_rows=8192)
    return out_packed.reshape(batch, fields, _OUT)


# speedup vs baseline: 1.1384x; 1.0008x over previous
import functools

import jax
import jax.numpy as jnp
from jax import lax
from jax.experimental import pallas as pl
from jax.experimental.pallas import tpu as pltpu
from jax.experimental.pallas import tpu_sc as plsc

_HIDDEN = 32
_OUT = 16
_PACK = 8
_NC, _NS = 2, 16


def _sc_gather(table, idx):
    num_idx = idx.shape[0]
    d = table.shape[1]
    nw = _NC * _NS
    b_per_w = num_idx // nw
    chunk = 3328

    mesh = plsc.VectorSubcoreMesh(core_axis_name="c", subcore_axis_name="s")

    @functools.partial(
        pl.kernel,
        mesh=mesh,
        out_type=jax.ShapeDtypeStruct((num_idx, d), jnp.float32),
        compiler_params=pltpu.CompilerParams(use_tc_tiling_on_sc=False),
        scratch_types=[
            pltpu.VMEM((b_per_w,), jnp.int32),
            pltpu.VMEM((chunk, d), jnp.float32),
            pltpu.SemaphoreType.DMA,
        ],
    )
    def gather_kernel(table_hbm, idx_hbm, out_hbm, idx_v, rows_v, sem):
        wid = lax.axis_index("s") * _NC + lax.axis_index("c")
        base = wid * b_per_w
        pltpu.sync_copy(idx_hbm.at[pl.ds(base, b_per_w)], idx_v)

        @pl.loop(0, b_per_w, step=chunk)
        def _(i):
            pltpu.async_copy(
                table_hbm.at[idx_v.at[pl.ds(i, chunk)]], rows_v, sem
            ).wait()
            pltpu.sync_copy(rows_v, out_hbm.at[pl.ds(base + i, chunk)])

    return gather_kernel(table, idx)


def _project_body(g_ref, wbig_ref, bbig_ref, out_ref):
    out_ref[...] = (
        jnp.dot(g_ref[...], wbig_ref[...], preferred_element_type=jnp.float32)
        + bbig_ref[...]
    )


def _project(g_packed, w_big, b_big, block_rows):
    n_rows = g_packed.shape[0]
    grid = (n_rows // block_rows,)
    return pl.pallas_call(
        _project_body,
        grid=grid,
        in_specs=[
            pl.BlockSpec((block_rows, _PACK * _HIDDEN), lambda i: (i, 0)),
            pl.BlockSpec((_PACK * _HIDDEN, _PACK * _OUT), lambda i: (0, 0)),
            pl.BlockSpec((1, _PACK * _OUT), lambda i: (0, 0)),
        ],
        out_specs=pl.BlockSpec((block_rows, _PACK * _OUT), lambda i: (i, 0)),
        out_shape=jax.ShapeDtypeStruct((n_rows, _PACK * _OUT), jnp.float32),
    )(g_packed, w_big, b_big)


def kernel(x, emb, W, b):
    batch, fields = x.shape
    idx = x.reshape(-1).astype(jnp.int32)
    g = _sc_gather(emb, idx)
    g_packed = g.reshape(batch * fields // _PACK, _PACK * _HIDDEN)
    w_big = jnp.kron(jnp.eye(_PACK, dtype=W.dtype), W)
    b_big = jnp.tile(b, _PACK).reshape(1, _PACK * _OUT)
    out_packed = _project(g_packed, w_big, b_big, block_rows=6656)
    return out_packed.reshape(batch, fields, _OUT)


# speedup vs baseline: 1.4638x; 1.2859x over previous
import functools

import jax
import jax.numpy as jnp
from jax import lax
from jax.experimental import pallas as pl
from jax.experimental.layout import Format, Layout, with_layout_constraint
from jax.experimental.pallas import tpu as pltpu
from jax.experimental.pallas import tpu_sc as plsc

_HIDDEN = 32
_OUT = 16
_PACK = 8
_NC, _NS = 2, 16


def _sc_gather(table, idx):
    num_idx = idx.shape[0]
    d = table.shape[1]
    nw = _NC * _NS
    b_per_w = num_idx // nw
    chunk = 3328

    mesh = plsc.VectorSubcoreMesh(core_axis_name="c", subcore_axis_name="s")

    @functools.partial(
        pl.kernel,
        mesh=mesh,
        out_type=jax.ShapeDtypeStruct((num_idx, d), jnp.float32),
        compiler_params=pltpu.CompilerParams(use_tc_tiling_on_sc=False),
        scratch_types=[
            pltpu.VMEM((b_per_w,), jnp.int32),
            pltpu.VMEM((chunk, d), jnp.float32),
            pltpu.SemaphoreType.DMA,
        ],
    )
    def gather_kernel(table_hbm, idx_hbm, out_hbm, idx_v, rows_v, sem):
        wid = lax.axis_index("s") * _NC + lax.axis_index("c")
        base = wid * b_per_w
        pltpu.sync_copy(idx_hbm.at[pl.ds(base, b_per_w)], idx_v)

        @pl.loop(0, b_per_w, step=chunk)
        def _(i):
            pltpu.async_copy(
                table_hbm.at[idx_v.at[pl.ds(i, chunk)]], rows_v, sem
            ).wait()
            pltpu.sync_copy(rows_v, out_hbm.at[pl.ds(base + i, chunk)])

    return gather_kernel(table, idx)


def _project_body(g_ref, wbig_ref, bbig_ref, out_ref):
    out_ref[...] = (
        jnp.dot(g_ref[...], wbig_ref[...], preferred_element_type=jnp.float32)
        + bbig_ref[...]
    )


def _project(g_packed, w_big, b_big, block_rows):
    n_rows = g_packed.shape[0]
    grid = (n_rows // block_rows,)
    return pl.pallas_call(
        _project_body,
        grid=grid,
        in_specs=[
            pl.BlockSpec((block_rows, _PACK * _HIDDEN), lambda i: (i, 0)),
            pl.BlockSpec((_PACK * _HIDDEN, _PACK * _OUT), lambda i: (0, 0)),
            pl.BlockSpec((1, _PACK * _OUT), lambda i: (0, 0)),
        ],
        out_specs=pl.BlockSpec((block_rows, _PACK * _OUT), lambda i: (i, 0)),
        out_shape=jax.ShapeDtypeStruct((n_rows, _PACK * _OUT), jnp.float32),
    )(g_packed, w_big, b_big)


def kernel(x, emb, W, b):
    batch, fields = x.shape
    idx = x.reshape(-1).astype(jnp.int32)
    emb_lin = with_layout_constraint(
        emb, Layout(major_to_minor=(0, 1), tiling=((16,),))
    )
    g = _sc_gather(emb_lin, idx)
    g_packed = g.reshape(batch * fields // _PACK, _PACK * _HIDDEN)
    w_big = jnp.kron(jnp.eye(_PACK, dtype=W.dtype), W)
    b_big = jnp.tile(b, _PACK).reshape(1, _PACK * _OUT)
    out_packed = _project(g_packed, w_big, b_big, block_rows=6656)
    return out_packed.reshape(batch, fields, _OUT)
